# baseline (speedup 1.0000x reference)
import functools

import jax
import jax.numpy as jnp
import numpy as np
from jax import lax
from jax.experimental import pallas as pl
from jax.experimental.pallas import tpu as pltpu
from jax.experimental.pallas import tpu_sc as plsc

N_NODES = 10000
N_EDGES = 160000
D = 256

NC = 2
NS = 16
NW = NC * NS

E_A = 81920
E_B = N_EDGES - E_A

_sc_mesh = lambda: plsc.VectorSubcoreMesh(core_axis_name="c", subcore_axis_name="s")

G_CH = 128


def _make_gather(e_off, e_len):
    epw = e_len // NW
    g_nf = epw // G_CH
    g_tail = epw - g_nf * G_CH

    def body(x_hbm, src_hbm, out_hbm, idx_v, rows0, rows1, gs0, gs1, ws0, ws1):
        wid = lax.axis_index("s") * NC + lax.axis_index("c")
        base = wid * epw
        pltpu.sync_copy(src_hbm.at[pl.ds(e_off + base, epw)], idx_v)
        rows = (rows0, rows1)
        gsem = (gs0, gs1)
        wsem = (ws0, ws1)

        def idx_slice(jj):
            return idx_v.at[pl.ds(pl.multiple_of(jj * G_CH, 8), G_CH)]

        def out_slice(jj):
            return out_hbm.at[pl.ds(pl.multiple_of(base + jj * G_CH, 8), G_CH)]

        def pair(j2, carry):
            for b in (0, 1):
                jj = j2 * 2 + b
                nb = 1 - b

                @pl.when(jj < g_nf)
                def _():
                    @pl.when(jj >= 2)
                    def _():
                        pltpu.make_async_copy(rows[b], out_slice(0),
                                              wsem[b]).wait()

                    pltpu.async_copy(x_hbm.at[idx_slice(jj)], rows[b], gsem[b])

                @pl.when((jj >= 1) & (jj <= g_nf))
                def _():
                    pltpu.make_async_copy(
                        x_hbm.at[pl.ds(0, G_CH)], rows[nb], gsem[nb]).wait()
                    pltpu.make_async_copy(rows[nb], out_slice(jj - 1),
                                          wsem[nb]).start()

            return carry

        lax.fori_loop(0, (g_nf + 2) // 2, pair, 0)
        pltpu.make_async_copy(rows[0], out_slice(0), wsem[0]).wait()
        pltpu.make_async_copy(rows[1], out_slice(0), wsem[1]).wait()
        if g_tail:
            toff = g_nf * G_CH
            pltpu.sync_copy(x_hbm.at[idx_v.at[pl.ds(toff, g_tail)]],
                            rows0.at[pl.ds(0, g_tail)])
            pltpu.sync_copy(rows0.at[pl.ds(0, g_tail)],
                            out_hbm.at[pl.ds(base + toff, g_tail)])

    def run(x_feat, src):
        k = pl.kernel(
            body,
            out_type=jax.ShapeDtypeStruct((e_len, D), jnp.float32),
            mesh=_sc_mesh(),
            scratch_types=[
                pltpu.VMEM((epw,), jnp.int32),
                pltpu.VMEM((G_CH, D), jnp.float32),
                pltpu.VMEM((G_CH, D), jnp.float32),
                pltpu.SemaphoreType.DMA,
                pltpu.SemaphoreType.DMA,
                pltpu.SemaphoreType.DMA,
                pltpu.SemaphoreType.DMA,
            ],
        )
        return k(x_feat, src)

    return run


HALF = N_NODES // NC
N_GARB = 64
S_CH = 80
R_CH = 8
N_RCH = HALF // R_CH
RCH_PER_T = -(-N_RCH // NS)


def _make_scatter(e_off, e_len):
    ept = e_len // NS
    s_nch = ept // S_CH

    def body(h_hbm, dst_hbm, x_hbm, out_hbm, aggr_sh, db0, db1, ib0, ib1,
             rows0, rows1, zbuf, abuf, xbuf, gs0, gs1, ss0, ss1):
        c = lax.axis_index("c")
        t = lax.axis_index("s")
        lo = c * HALF
        ebase = t * ept
        lanes = lax.iota(jnp.int32, 16)
        zero = jnp.zeros((16,), jnp.float32)

        for r in range(R_CH):
            for q in range(D // 16):
                zbuf[r, pl.ds(q * 16, 16)] = zero

        def init_chunk(i, carry):
            cid = t + i * NS

            @pl.when(cid < N_RCH)
            def _():
                off = pl.multiple_of(cid * R_CH, 8)
                pltpu.sync_copy(zbuf, aggr_sh.at[pl.ds(off, R_CH)])

            return carry

        lax.fori_loop(0, RCH_PER_T, init_chunk, 0)
        plsc.subcore_barrier()

        dbuf = (db0, db1)
        ibuf = (ib0, ib1)
        rows = (rows0, rows1)
        gsem = (gs0, gs1)
        ssem = (ss0, ss1)

        def compute_idx(db, ib, jj):
            for q in range(S_CH // 16):
                d = db[pl.ds(q * 16, 16)] - lo
                m = (d >= 0) & (d < HALF)
                garb = HALF + ((lanes + jj + q) & (N_GARB - 1))
                ib[pl.ds(q * 16, 16)] = jnp.where(m, d, garb)

        def pair(j2, carry):
            for b in (0, 1):
                jj = j2 * 2 + b
                nb = 1 - b

                @pl.when(jj < s_nch)
                def _():
                    @pl.when(jj >= 2)
                    def _():
                        pltpu.make_async_copy(
                            rows[b], aggr_sh.at[pl.ds(0, S_CH)], ssem[b]).wait()

                    eoff = pl.multiple_of(ebase + jj * S_CH, 8)
                    pltpu.sync_copy(dst_hbm.at[pl.ds(e_off + eoff, S_CH)],
                                    dbuf[b])
                    compute_idx(dbuf[b], ibuf[b], jj)
                    pltpu.async_copy(
                        h_hbm.at[pl.ds(eoff, S_CH)], rows[b], gsem[b])

                @pl.when((jj >= 1) & (jj <= s_nch))
                def _():
                    pltpu.make_async_copy(
                        h_hbm.at[pl.ds(0, S_CH)], rows[nb], gsem[nb]).wait()
                    pltpu.make_async_copy(
                        rows[nb], aggr_sh.at[ibuf[nb]], ssem[nb]
                    ).start(add=True)

            return carry

        lax.fori_loop(0, (s_nch + 2) // 2, pair, 0)
        pltpu.make_async_copy(rows[0], aggr_sh.at[pl.ds(0, S_CH)],
                              ssem[0]).wait()
        pltpu.make_async_copy(rows[1], aggr_sh.at[pl.ds(0, S_CH)],
                              ssem[1]).wait()

        plsc.subcore_barrier()

        def out_chunk(i, carry):
            cid = t + i * NS

            @pl.when(cid < N_RCH)
            def _():
                off = pl.multiple_of(cid * R_CH, 8)
                goff = pl.multiple_of(lo + cid * R_CH, 8)
                pltpu.sync_copy(aggr_sh.at[pl.ds(off, R_CH)], abuf)
                pltpu.sync_copy(x_hbm.at[pl.ds(goff, R_CH)], xbuf)
                for r in range(R_CH):
                    for q in range(D // 16):
                        sl = pl.ds(q * 16, 16)
                        abuf[r, sl] = abuf[r, sl] + xbuf[r, sl]
                pltpu.sync_copy(abuf, out_hbm.at[pl.ds(goff, R_CH)])

            return carry

        lax.fori_loop(0, RCH_PER_T, out_chunk, 0)

    def run(h, dst, x_prev):
        k = pl.kernel(
            body,
            out_type=jax.ShapeDtypeStruct((N_NODES, D), jnp.float32),
            mesh=_sc_mesh(),
            compiler_params=pltpu.CompilerParams(use_tc_tiling_on_sc=False),
            scratch_types=[
                pltpu.VMEM_SHARED((HALF + N_GARB, D), jnp.float32),
                pltpu.VMEM((S_CH,), jnp.int32),
                pltpu.VMEM((S_CH,), jnp.int32),
                pltpu.VMEM((S_CH,), jnp.int32),
                pltpu.VMEM((S_CH,), jnp.int32),
                pltpu.VMEM((S_CH, D), jnp.float32),
                pltpu.VMEM((S_CH, D), jnp.float32),
                pltpu.VMEM((R_CH, D), jnp.float32),
                pltpu.VMEM((R_CH, D), jnp.float32),
                pltpu.VMEM((R_CH, D), jnp.float32),
                pltpu.SemaphoreType.DMA,
                pltpu.SemaphoreType.DMA,
                pltpu.SemaphoreType.DMA,
                pltpu.SemaphoreType.DMA,
            ],
        )
        return k(h, dst, x_prev)

    return run


BE = 640


def _edge_ffn_body(xg_ref, ea_ref, bs_ref, w_ref, b_ref, o_ref):
    xe = xg_ref[...] + ea_ref[...]
    z = jnp.dot(xe, w_ref[...], preferred_element_type=jnp.float32) + b_ref[...]
    g = 0.5 * z * (1.0 + lax.erf(z * np.float32(1.0 / np.sqrt(2.0))))
    o_ref[...] = g * bs_ref[...]


def _make_edge_ffn(e_off, e_len):
    ob = e_off // BE

    def run(xg, edge_attr, bases, W_pre, b_pre):
        return pl.pallas_call(
            _edge_ffn_body,
            grid=(e_len // BE,),
            in_specs=[
                pl.BlockSpec((BE, D), lambda i: (i, 0)),
                pl.BlockSpec((BE, D), lambda i: (i + ob, 0)),
                pl.BlockSpec((BE, D), lambda i: (i + ob, 0)),
                pl.BlockSpec((D, D), lambda i: (0, 0)),
                pl.BlockSpec((1, D), lambda i: (0, 0)),
            ],
            out_specs=pl.BlockSpec((BE, D), lambda i: (i, 0)),
            out_shape=jax.ShapeDtypeStruct((e_len, D), jnp.float32),
        )(xg, edge_attr, bases, W_pre, b_pre.reshape(1, D))

    return run


def _node_ffn_body(x_ref, w1_ref, b1_ref, g1_ref, be1_ref, w2_ref, b2_ref,
                   g2_ref, be2_ref, o_ref):
    x = x_ref[...]
    y = jnp.dot(x, w1_ref[...], preferred_element_type=jnp.float32) + b1_ref[...]
    m = jnp.mean(y, axis=0, keepdims=True)
    v = jnp.mean((y - m) * (y - m), axis=0, keepdims=True)
    y = (y - m) * lax.rsqrt(v + 1e-5) * g1_ref[...] + be1_ref[...]
    y = jnp.maximum(y, 0.0)
    y = jnp.dot(y, w2_ref[...], preferred_element_type=jnp.float32) + b2_ref[...]
    m = jnp.mean(y, axis=0, keepdims=True)
    v = jnp.mean((y - m) * (y - m), axis=0, keepdims=True)
    y = (y - m) * lax.rsqrt(v + 1e-5) * g2_ref[...] + be2_ref[...]
    y = jnp.maximum(y, 0.0)
    o_ref[...] = x + y


def _tc_node_ffn(x, W1, b1, g1, be1, W2, b2, g2, be2):
    row = lambda a: a.reshape(1, D)
    return pl.pallas_call(
        _node_ffn_body,
        out_shape=jax.ShapeDtypeStruct((N_NODES, D), jnp.float32),
    )(x, W1, row(b1), row(g1), row(be1), W2, row(b2), row(g2), row(be2))


def kernel(x_feat, edge_index, edge_attr, bases, W_pre, b_pre, W1, b1, g1, be1,
           W2, b2, g2, be2):
    src = edge_index[0]
    dst = edge_index[1]
    gather_a = _make_gather(0, E_A)
    gather_b = _make_gather(E_A, E_B)
    ffn_a = _make_edge_ffn(0, E_A)
    ffn_b = _make_edge_ffn(E_A, E_B)
    scatter_a = _make_scatter(0, E_A)
    scatter_b = _make_scatter(E_A, E_B)

    xg_a = gather_a(x_feat, src)
    xg_b = gather_b(x_feat, src)
    h_a = ffn_a(xg_a, edge_attr, bases, W_pre, b_pre)
    x1 = scatter_a(h_a, dst, x_feat)
    h_b = ffn_b(xg_b, edge_attr, bases, W_pre, b_pre)
    x2 = scatter_b(h_b, dst, x1)
    return _tc_node_ffn(x2, W1, b1, g1, be1, W2, b2, g2, be2)


# speedup vs baseline: 1.1565x; 1.1565x over previous
import functools

import jax
import jax.numpy as jnp
import numpy as np
from jax import lax
from jax.experimental import pallas as pl
from jax.experimental.pallas import tpu as pltpu
from jax.experimental.pallas import tpu_sc as plsc

N_NODES = 10000
N_EDGES = 160000
D = 256

NC = 2
NS = 16
NW = NC * NS

_sc_mesh = lambda: plsc.VectorSubcoreMesh(core_axis_name="c", subcore_axis_name="s")

E_PER_W = N_EDGES // NW
G_CH = 128
G_NF = E_PER_W // G_CH
G_TAIL = E_PER_W - G_NF * G_CH


def _gather_body(x_hbm, src_hbm, out_hbm, idx_v, rows0, rows1, gs0, gs1,
                 ws0, ws1):
    wid = lax.axis_index("s") * NC + lax.axis_index("c")
    base = wid * E_PER_W
    pltpu.sync_copy(src_hbm.at[pl.ds(base, E_PER_W)], idx_v)
    rows = (rows0, rows1)
    gsem = (gs0, gs1)
    wsem = (ws0, ws1)

    def idx_slice(jj):
        return idx_v.at[pl.ds(pl.multiple_of(jj * G_CH, 8), G_CH)]

    def out_slice(jj):
        return out_hbm.at[pl.ds(pl.multiple_of(base + jj * G_CH, 8), G_CH)]

    def pair(j2, carry):
        for b in (0, 1):
            jj = j2 * 2 + b
            nb = 1 - b

            @pl.when(jj < G_NF)
            def _():
                @pl.when(jj >= 2)
                def _():
                    pltpu.make_async_copy(rows[b], out_slice(0), wsem[b]).wait()

                pltpu.async_copy(x_hbm.at[idx_slice(jj)], rows[b], gsem[b])

            @pl.when((jj >= 1) & (jj <= G_NF))
            def _():
                pltpu.make_async_copy(
                    x_hbm.at[pl.ds(0, G_CH)], rows[nb], gsem[nb]).wait()
                pltpu.make_async_copy(rows[nb], out_slice(jj - 1),
                                      wsem[nb]).start()

        return carry

    lax.fori_loop(0, (G_NF + 2) // 2, pair, 0)
    pltpu.make_async_copy(rows[0], out_slice(0), wsem[0]).wait()
    pltpu.make_async_copy(rows[1], out_slice(0), wsem[1]).wait()
    toff = G_NF * G_CH
    pltpu.sync_copy(x_hbm.at[idx_v.at[pl.ds(toff, G_TAIL)]],
                    rows0.at[pl.ds(0, G_TAIL)])
    pltpu.sync_copy(rows0.at[pl.ds(0, G_TAIL)],
                    out_hbm.at[pl.ds(base + toff, G_TAIL)])


def _sc_gather(x_feat, src):
    k = pl.kernel(
        _gather_body,
        out_type=jax.ShapeDtypeStruct((N_EDGES, D), jnp.float32),
        mesh=_sc_mesh(),
        scratch_types=[
            pltpu.VMEM((E_PER_W,), jnp.int32),
            pltpu.VMEM((G_CH, D), jnp.float32),
            pltpu.VMEM((G_CH, D), jnp.float32),
            pltpu.SemaphoreType.DMA,
            pltpu.SemaphoreType.DMA,
            pltpu.SemaphoreType.DMA,
            pltpu.SemaphoreType.DMA,
        ],
    )
    return k(x_feat, src)


HALF = N_NODES // NC
N_GARB = 64
E_PER_T = N_EDGES // NS
S_CH = 80
S_NCH = E_PER_T // S_CH
R_CH = 8
N_RCH = HALF // R_CH
RCH_PER_T = -(-N_RCH // NS)


def _scatter_body(h_hbm, dst_hbm, x_hbm, out_hbm, aggr_sh, db0, db1, ib0, ib1,
                  rows0, rows1, zbuf, abuf, xbuf, gs0, gs1, ss0, ss1, ds0, ds1):
    c = lax.axis_index("c")
    t = lax.axis_index("s")
    lo = c * HALF
    ebase = t * E_PER_T
    lanes = lax.iota(jnp.int32, 16)
    zero = jnp.zeros((16,), jnp.float32)

    for r in range(R_CH):
        for q in range(D // 16):
            zbuf[r, pl.ds(q * 16, 16)] = zero

    def init_chunk(i, carry):
        cid = t + i * NS

        @pl.when(cid < N_RCH)
        def _():
            off = pl.multiple_of(cid * R_CH, 8)
            pltpu.sync_copy(zbuf, aggr_sh.at[pl.ds(off, R_CH)])

        return carry

    lax.fori_loop(0, RCH_PER_T, init_chunk, 0)
    plsc.subcore_barrier()

    dbuf = (db0, db1)
    ibuf = (ib0, ib1)
    rows = (rows0, rows1)
    gsem = (gs0, gs1)
    ssem = (ss0, ss1)
    dsem = (ds0, ds1)

    def compute_idx(db, ib, jj):
        for q in range(S_CH // 16):
            d = db[pl.ds(q * 16, 16)] - lo
            m = (d >= 0) & (d < HALF)
            garb = HALF + ((lanes + jj + q) & (N_GARB - 1))
            ib[pl.ds(q * 16, 16)] = jnp.where(m, d, garb)

    def dst_slice(jj):
        return dst_hbm.at[pl.ds(pl.multiple_of(ebase + jj * S_CH, 8), S_CH)]

    pltpu.async_copy(dst_slice(0), dbuf[0], dsem[0])
    pltpu.async_copy(dst_slice(1), dbuf[1], dsem[1])

    def pair(j2, carry):
        for b in (0, 1):
            jj = j2 * 2 + b
            nb = 1 - b

            @pl.when(jj < S_NCH)
            def _():
                @pl.when(jj >= 2)
                def _():
                    pltpu.make_async_copy(
                        rows[b], aggr_sh.at[pl.ds(0, S_CH)], ssem[b]).wait()

                pltpu.make_async_copy(
                    dst_slice(0), dbuf[b], dsem[b]).wait()
                compute_idx(dbuf[b], ibuf[b], jj)

                @pl.when(jj + 2 < S_NCH)
                def _():
                    pltpu.async_copy(dst_slice(jj + 2), dbuf[b], dsem[b])

                eoff = pl.multiple_of(ebase + jj * S_CH, 8)
                pltpu.async_copy(
                    h_hbm.at[pl.ds(eoff, S_CH)], rows[b], gsem[b])

            @pl.when((jj >= 1) & (jj <= S_NCH))
            def _():
                pltpu.make_async_copy(
                    h_hbm.at[pl.ds(0, S_CH)], rows[nb], gsem[nb]).wait()
                pltpu.make_async_copy(
                    rows[nb], aggr_sh.at[ibuf[nb]], ssem[nb]).start(add=True)

        return carry

    lax.fori_loop(0, (S_NCH + 2) // 2, pair, 0)
    pltpu.make_async_copy(rows[0], aggr_sh.at[pl.ds(0, S_CH)], ssem[0]).wait()
    pltpu.make_async_copy(rows[1], aggr_sh.at[pl.ds(0, S_CH)], ssem[1]).wait()

    plsc.subcore_barrier()

    def out_chunk(i, carry):
        cid = t + i * NS

        @pl.when(cid < N_RCH)
        def _():
            off = pl.multiple_of(cid * R_CH, 8)
            goff = pl.multiple_of(lo + cid * R_CH, 8)
            pltpu.sync_copy(aggr_sh.at[pl.ds(off, R_CH)], abuf)
            pltpu.sync_copy(x_hbm.at[pl.ds(goff, R_CH)], xbuf)
            for r in range(R_CH):
                for q in range(D // 16):
                    sl = pl.ds(q * 16, 16)
                    abuf[r, sl] = abuf[r, sl] + xbuf[r, sl]
            pltpu.sync_copy(abuf, out_hbm.at[pl.ds(goff, R_CH)])

        return carry

    lax.fori_loop(0, RCH_PER_T, out_chunk, 0)


def _sc_scatter(h, dst, x_feat):
    k = pl.kernel(
        _scatter_body,
        out_type=jax.ShapeDtypeStruct((N_NODES, D), jnp.float32),
        mesh=_sc_mesh(),
        compiler_params=pltpu.CompilerParams(use_tc_tiling_on_sc=False),
        scratch_types=[
            pltpu.VMEM_SHARED((HALF + N_GARB, D), jnp.float32),
            pltpu.VMEM((S_CH,), jnp.int32),
            pltpu.VMEM((S_CH,), jnp.int32),
            pltpu.VMEM((S_CH,), jnp.int32),
            pltpu.VMEM((S_CH,), jnp.int32),
            pltpu.VMEM((S_CH, D), jnp.float32),
            pltpu.VMEM((S_CH, D), jnp.float32),
            pltpu.VMEM((R_CH, D), jnp.float32),
            pltpu.VMEM((R_CH, D), jnp.float32),
            pltpu.VMEM((R_CH, D), jnp.float32),
            pltpu.SemaphoreType.DMA,
            pltpu.SemaphoreType.DMA,
            pltpu.SemaphoreType.DMA,
            pltpu.SemaphoreType.DMA,
            pltpu.SemaphoreType.DMA,
            pltpu.SemaphoreType.DMA,
        ],
    )
    return k(h, dst, x_feat)


BE = 1000


def _edge_ffn_body(xg_ref, ea_ref, bs_ref, w_ref, b_ref, o_ref):
    xe = xg_ref[...] + ea_ref[...]
    z = jnp.dot(xe, w_ref[...], preferred_element_type=jnp.float32) + b_ref[...]
    g = 0.5 * z * (1.0 + lax.erf(z * np.float32(1.0 / np.sqrt(2.0))))
    o_ref[...] = g * bs_ref[...]


def _tc_edge_ffn(xg, edge_attr, bases, W_pre, b_pre):
    return pl.pallas_call(
        _edge_ffn_body,
        grid=(N_EDGES // BE,),
        in_specs=[
            pl.BlockSpec((BE, D), lambda i: (i, 0)),
            pl.BlockSpec((BE, D), lambda i: (i, 0)),
            pl.BlockSpec((BE, D), lambda i: (i, 0)),
            pl.BlockSpec((D, D), lambda i: (0, 0)),
            pl.BlockSpec((1, D), lambda i: (0, 0)),
        ],
        out_specs=pl.BlockSpec((BE, D), lambda i: (i, 0)),
        out_shape=jax.ShapeDtypeStruct((N_EDGES, D), jnp.float32),
    )(xg, edge_attr, bases, W_pre, b_pre.reshape(1, D))


def _node_ffn_body(x_ref, w1_ref, b1_ref, g1_ref, be1_ref, w2_ref, b2_ref,
                   g2_ref, be2_ref, o_ref):
    x = x_ref[...]
    y = jnp.dot(x, w1_ref[...], preferred_element_type=jnp.float32) + b1_ref[...]
    m = jnp.mean(y, axis=0, keepdims=True)
    v = jnp.mean((y - m) * (y - m), axis=0, keepdims=True)
    y = (y - m) * lax.rsqrt(v + 1e-5) * g1_ref[...] + be1_ref[...]
    y = jnp.maximum(y, 0.0)
    y = jnp.dot(y, w2_ref[...], preferred_element_type=jnp.float32) + b2_ref[...]
    m = jnp.mean(y, axis=0, keepdims=True)
    v = jnp.mean((y - m) * (y - m), axis=0, keepdims=True)
    y = (y - m) * lax.rsqrt(v + 1e-5) * g2_ref[...] + be2_ref[...]
    y = jnp.maximum(y, 0.0)
    o_ref[...] = x + y


def _tc_node_ffn(x, W1, b1, g1, be1, W2, b2, g2, be2):
    row = lambda a: a.reshape(1, D)
    return pl.pallas_call(
        _node_ffn_body,
        out_shape=jax.ShapeDtypeStruct((N_NODES, D), jnp.float32),
    )(x, W1, row(b1), row(g1), row(be1), W2, row(b2), row(g2), row(be2))


def kernel(x_feat, edge_index, edge_attr, bases, W_pre, b_pre, W1, b1, g1, be1,
           W2, b2, g2, be2):
    src = edge_index[0]
    dst = edge_index[1]
    xg = _sc_gather(x_feat, src)
    h = _tc_edge_ffn(xg, edge_attr, bases, W_pre, b_pre)
    x = _sc_scatter(h, dst, x_feat)
    return _tc_node_ffn(x, W1, b1, g1, be1, W2, b2, g2, be2)


# speedup vs baseline: 1.2152x; 1.0507x over previous
import functools

import jax
import jax.numpy as jnp
import numpy as np
from jax import lax
from jax.experimental import pallas as pl
from jax.experimental.pallas import tpu as pltpu
from jax.experimental.pallas import tpu_sc as plsc

N_NODES = 10000
N_EDGES = 160000
D = 256

NC = 2
NS = 16
NW = NC * NS

_sc_mesh = lambda: plsc.VectorSubcoreMesh(core_axis_name="c", subcore_axis_name="s")

E_PER_W = N_EDGES // NW
G_CH = 128
G_NF = E_PER_W // G_CH
G_TAIL = E_PER_W - G_NF * G_CH


def _gather_body(x_hbm, src_hbm, out_hbm, idx_v, rows0, rows1, gs0, gs1,
                 ws0, ws1):
    wid = lax.axis_index("s") * NC + lax.axis_index("c")
    base = wid * E_PER_W
    pltpu.sync_copy(src_hbm.at[pl.ds(base, E_PER_W)], idx_v)
    rows = (rows0, rows1)
    gsem = (gs0, gs1)
    wsem = (ws0, ws1)

    def idx_slice(jj):
        return idx_v.at[pl.ds(pl.multiple_of(jj * G_CH, 8), G_CH)]

    def out_slice(jj):
        return out_hbm.at[pl.ds(pl.multiple_of(base + jj * G_CH, 8), G_CH)]

    def pair(j2, carry):
        for b in (0, 1):
            jj = j2 * 2 + b
            nb = 1 - b

            @pl.when(jj < G_NF)
            def _():
                @pl.when(jj >= 2)
                def _():
                    pltpu.make_async_copy(rows[b], out_slice(0), wsem[b]).wait()

                pltpu.async_copy(x_hbm.at[idx_slice(jj)], rows[b], gsem[b])

            @pl.when((jj >= 1) & (jj <= G_NF))
            def _():
                pltpu.make_async_copy(
                    x_hbm.at[pl.ds(0, G_CH)], rows[nb], gsem[nb]).wait()
                pltpu.make_async_copy(rows[nb], out_slice(jj - 1),
                                      wsem[nb]).start()

        return carry

    lax.fori_loop(0, (G_NF + 2) // 2, pair, 0)
    pltpu.make_async_copy(rows[0], out_slice(0), wsem[0]).wait()
    pltpu.make_async_copy(rows[1], out_slice(0), wsem[1]).wait()
    toff = G_NF * G_CH
    pltpu.sync_copy(x_hbm.at[idx_v.at[pl.ds(toff, G_TAIL)]],
                    rows0.at[pl.ds(0, G_TAIL)])
    pltpu.sync_copy(rows0.at[pl.ds(0, G_TAIL)],
                    out_hbm.at[pl.ds(base + toff, G_TAIL)])


def _sc_gather(x_feat, src):
    k = pl.kernel(
        _gather_body,
        out_type=jax.ShapeDtypeStruct((N_EDGES, D), jnp.float32),
        mesh=_sc_mesh(),
        scratch_types=[
            pltpu.VMEM((E_PER_W,), jnp.int32),
            pltpu.VMEM((G_CH, D), jnp.float32),
            pltpu.VMEM((G_CH, D), jnp.float32),
            pltpu.SemaphoreType.DMA,
            pltpu.SemaphoreType.DMA,
            pltpu.SemaphoreType.DMA,
            pltpu.SemaphoreType.DMA,
        ],
    )
    return k(x_feat, src)


HALF = N_NODES // NC
N_GARB = 64
E_PER_T = N_EDGES // NS
S_CH = 80
S_NCH = E_PER_T // S_CH
R_CH = 8
N_RCH = HALF // R_CH
RCH_PER_T = -(-N_RCH // NS)


def _scatter_body(h_hbm, dst_hbm, x_hbm, out_hbm, aggr_sh, db0, db1, ib0, ib1,
                  rows0, rows1, zbuf, abuf, xbuf, gs0, gs1, ss0, ss1, ds0, ds1):
    c = lax.axis_index("c")
    t = lax.axis_index("s")
    lo = c * HALF
    ebase = t * E_PER_T
    lanes = lax.iota(jnp.int32, 16)
    zero = jnp.zeros((16,), jnp.float32)

    for r in range(R_CH):
        for q in range(D // 16):
            zbuf[r, pl.ds(q * 16, 16)] = zero

    def init_chunk(i, carry):
        cid = t + i * NS

        @pl.when(cid < N_RCH)
        def _():
            off = pl.multiple_of(cid * R_CH, 8)
            pltpu.sync_copy(zbuf, aggr_sh.at[pl.ds(off, R_CH)])

        return carry

    lax.fori_loop(0, RCH_PER_T, init_chunk, 0)
    plsc.subcore_barrier()

    dbuf = (db0, db1)
    ibuf = (ib0, ib1)
    rows = (rows0, rows1)
    gsem = (gs0, gs1)
    ssem = (ss0, ss1)
    dsem = (ds0, ds1)

    def compute_idx(db, ib, jj):
        for q in range(S_CH // 16):
            d = db[pl.ds(q * 16, 16)] - lo
            m = (d >= 0) & (d < HALF)
            garb = HALF + ((lanes + jj + q) & (N_GARB - 1))
            ib[pl.ds(q * 16, 16)] = jnp.where(m, d, garb)

    def dst_slice(jj):
        return dst_hbm.at[pl.ds(pl.multiple_of(ebase + jj * S_CH, 8), S_CH)]

    pltpu.async_copy(dst_slice(0), dbuf[0], dsem[0])
    pltpu.async_copy(dst_slice(1), dbuf[1], dsem[1])

    def pair(j2, carry):
        for b in (0, 1):
            jj = j2 * 2 + b
            nb = 1 - b

            @pl.when(jj < S_NCH)
            def _():
                @pl.when(jj >= 2)
                def _():
                    pltpu.make_async_copy(
                        rows[b], aggr_sh.at[pl.ds(0, S_CH)], ssem[b]).wait()

                eoff = pl.multiple_of(ebase + jj * S_CH, 8)
                pltpu.async_copy(
                    h_hbm.at[pl.ds(eoff, S_CH)], rows[b], gsem[b])
                pltpu.make_async_copy(
                    dst_slice(0), dbuf[b], dsem[b]).wait()
                compute_idx(dbuf[b], ibuf[b], jj)

                @pl.when(jj + 2 < S_NCH)
                def _():
                    pltpu.async_copy(dst_slice(jj + 2), dbuf[b], dsem[b])

            @pl.when((jj >= 1) & (jj <= S_NCH))
            def _():
                pltpu.make_async_copy(
                    h_hbm.at[pl.ds(0, S_CH)], rows[nb], gsem[nb]).wait()
                pltpu.make_async_copy(
                    rows[nb], aggr_sh.at[ibuf[nb]], ssem[nb]).start(add=True)

        return carry

    lax.fori_loop(0, (S_NCH + 2) // 2, pair, 0)
    pltpu.make_async_copy(rows[0], aggr_sh.at[pl.ds(0, S_CH)], ssem[0]).wait()
    pltpu.make_async_copy(rows[1], aggr_sh.at[pl.ds(0, S_CH)], ssem[1]).wait()

    plsc.subcore_barrier()

    def out_chunk(i, carry):
        cid = t + i * NS

        @pl.when(cid < N_RCH)
        def _():
            off = pl.multiple_of(cid * R_CH, 8)
            goff = pl.multiple_of(lo + cid * R_CH, 8)
            pltpu.sync_copy(aggr_sh.at[pl.ds(off, R_CH)], abuf)
            pltpu.sync_copy(x_hbm.at[pl.ds(goff, R_CH)], xbuf)
            for r in range(R_CH):
                for q in range(D // 16):
                    sl = pl.ds(q * 16, 16)
                    abuf[r, sl] = abuf[r, sl] + xbuf[r, sl]
            pltpu.sync_copy(abuf, out_hbm.at[pl.ds(goff, R_CH)])

        return carry

    lax.fori_loop(0, RCH_PER_T, out_chunk, 0)


def _sc_scatter(h, dst, x_feat):
    k = pl.kernel(
        _scatter_body,
        out_type=jax.ShapeDtypeStruct((N_NODES, D), jnp.float32),
        mesh=_sc_mesh(),
        compiler_params=pltpu.CompilerParams(use_tc_tiling_on_sc=False),
        scratch_types=[
            pltpu.VMEM_SHARED((HALF + N_GARB, D), jnp.float32),
            pltpu.VMEM((S_CH,), jnp.int32),
            pltpu.VMEM((S_CH,), jnp.int32),
            pltpu.VMEM((S_CH,), jnp.int32),
            pltpu.VMEM((S_CH,), jnp.int32),
            pltpu.VMEM((S_CH, D), jnp.float32),
            pltpu.VMEM((S_CH, D), jnp.float32),
            pltpu.VMEM((R_CH, D), jnp.float32),
            pltpu.VMEM((R_CH, D), jnp.float32),
            pltpu.VMEM((R_CH, D), jnp.float32),
            pltpu.SemaphoreType.DMA,
            pltpu.SemaphoreType.DMA,
            pltpu.SemaphoreType.DMA,
            pltpu.SemaphoreType.DMA,
            pltpu.SemaphoreType.DMA,
            pltpu.SemaphoreType.DMA,
        ],
    )
    return k(h, dst, x_feat)


BE = 2000


def _edge_ffn_body(xg_ref, ea_ref, bs_ref, w_ref, b_ref, o_ref):
    xe = xg_ref[...] + ea_ref[...]
    z = jnp.dot(xe, w_ref[...], preferred_element_type=jnp.float32) + b_ref[...]
    g = 0.5 * z * (1.0 + lax.erf(z * np.float32(1.0 / np.sqrt(2.0))))
    o_ref[...] = g * bs_ref[...]


def _tc_edge_ffn(xg, edge_attr, bases, W_pre, b_pre):
    return pl.pallas_call(
        _edge_ffn_body,
        grid=(N_EDGES // BE,),
        in_specs=[
            pl.BlockSpec((BE, D), lambda i: (i, 0)),
            pl.BlockSpec((BE, D), lambda i: (i, 0)),
            pl.BlockSpec((BE, D), lambda i: (i, 0)),
            pl.BlockSpec((D, D), lambda i: (0, 0)),
            pl.BlockSpec((1, D), lambda i: (0, 0)),
        ],
        out_specs=pl.BlockSpec((BE, D), lambda i: (i, 0)),
        out_shape=jax.ShapeDtypeStruct((N_EDGES, D), jnp.float32),
    )(xg, edge_attr, bases, W_pre, b_pre.reshape(1, D))


def _node_ffn_body(x_ref, w1_ref, b1_ref, g1_ref, be1_ref, w2_ref, b2_ref,
                   g2_ref, be2_ref, o_ref):
    x = x_ref[...]
    y = jnp.dot(x, w1_ref[...], preferred_element_type=jnp.float32) + b1_ref[...]
    m = jnp.mean(y, axis=0, keepdims=True)
    v = jnp.mean((y - m) * (y - m), axis=0, keepdims=True)
    y = (y - m) * lax.rsqrt(v + 1e-5) * g1_ref[...] + be1_ref[...]
    y = jnp.maximum(y, 0.0)
    y = jnp.dot(y, w2_ref[...], preferred_element_type=jnp.float32) + b2_ref[...]
    m = jnp.mean(y, axis=0, keepdims=True)
    v = jnp.mean((y - m) * (y - m), axis=0, keepdims=True)
    y = (y - m) * lax.rsqrt(v + 1e-5) * g2_ref[...] + be2_ref[...]
    y = jnp.maximum(y, 0.0)
    o_ref[...] = x + y


def _tc_node_ffn(x, W1, b1, g1, be1, W2, b2, g2, be2):
    row = lambda a: a.reshape(1, D)
    return pl.pallas_call(
        _node_ffn_body,
        out_shape=jax.ShapeDtypeStruct((N_NODES, D), jnp.float32),
    )(x, W1, row(b1), row(g1), row(be1), W2, row(b2), row(g2), row(be2))


def kernel(x_feat, edge_index, edge_attr, bases, W_pre, b_pre, W1, b1, g1, be1,
           W2, b2, g2, be2):
    src = edge_index[0]
    dst = edge_index[1]
    xg = _sc_gather(x_feat, src)
    h = _tc_edge_ffn(xg, edge_attr, bases, W_pre, b_pre)
    x = _sc_scatter(h, dst, x_feat)
    return _tc_node_ffn(x, W1, b1, g1, be1, W2, b2, g2, be2)


# speedup vs baseline: 1.2224x; 1.0059x over previous
import functools

import jax
import jax.numpy as jnp
import numpy as np
from jax import lax
from jax.experimental import pallas as pl
from jax.experimental.pallas import tpu as pltpu
from jax.experimental.pallas import tpu_sc as plsc

N_NODES = 10000
N_EDGES = 160000
D = 256

NC = 2
NS = 16
NW = NC * NS

_sc_mesh = lambda: plsc.VectorSubcoreMesh(core_axis_name="c", subcore_axis_name="s")

E_PER_W = N_EDGES // NW
G_CH = 128
G_NF = E_PER_W // G_CH
G_TAIL = E_PER_W - G_NF * G_CH


def _gather_body(x_hbm, src_hbm, out_hbm, idx_v, rows0, rows1, gs0, gs1,
                 ws0, ws1):
    wid = lax.axis_index("s") * NC + lax.axis_index("c")
    base = wid * E_PER_W
    pltpu.sync_copy(src_hbm.at[pl.ds(base, E_PER_W)], idx_v)
    rows = (rows0, rows1)
    gsem = (gs0, gs1)
    wsem = (ws0, ws1)

    def idx_slice(jj):
        return idx_v.at[pl.ds(pl.multiple_of(jj * G_CH, 8), G_CH)]

    def out_slice(jj):
        return out_hbm.at[pl.ds(pl.multiple_of(base + jj * G_CH, 8), G_CH)]

    def pair(j2, carry):
        for b in (0, 1):
            jj = j2 * 2 + b
            nb = 1 - b

            @pl.when(jj < G_NF)
            def _():
                @pl.when(jj >= 2)
                def _():
                    pltpu.make_async_copy(rows[b], out_slice(0), wsem[b]).wait()

                pltpu.async_copy(x_hbm.at[idx_slice(jj)], rows[b], gsem[b])

            @pl.when((jj >= 1) & (jj <= G_NF))
            def _():
                pltpu.make_async_copy(
                    x_hbm.at[pl.ds(0, G_CH)], rows[nb], gsem[nb]).wait()
                pltpu.make_async_copy(rows[nb], out_slice(jj - 1),
                                      wsem[nb]).start()

        return carry

    lax.fori_loop(0, (G_NF + 2) // 2, pair, 0)
    pltpu.make_async_copy(rows[0], out_slice(0), wsem[0]).wait()
    pltpu.make_async_copy(rows[1], out_slice(0), wsem[1]).wait()
    toff = G_NF * G_CH
    pltpu.sync_copy(x_hbm.at[idx_v.at[pl.ds(toff, G_TAIL)]],
                    rows0.at[pl.ds(0, G_TAIL)])
    pltpu.sync_copy(rows0.at[pl.ds(0, G_TAIL)],
                    out_hbm.at[pl.ds(base + toff, G_TAIL)])


def _sc_gather(x_feat, src):
    k = pl.kernel(
        _gather_body,
        out_type=jax.ShapeDtypeStruct((N_EDGES, D), jnp.float32),
        mesh=_sc_mesh(),
        scratch_types=[
            pltpu.VMEM((E_PER_W,), jnp.int32),
            pltpu.VMEM((G_CH, D), jnp.float32),
            pltpu.VMEM((G_CH, D), jnp.float32),
            pltpu.SemaphoreType.DMA,
            pltpu.SemaphoreType.DMA,
            pltpu.SemaphoreType.DMA,
            pltpu.SemaphoreType.DMA,
        ],
    )
    return k(x_feat, src)


HALF = N_NODES // NC
N_GARB = 64
E_PER_T = N_EDGES // NS
S_CH = 80
S_NCH = E_PER_T // S_CH
R_CH = 8
N_RCH = HALF // R_CH
RCH_PER_T = -(-N_RCH // NS)


def _scatter_body(h_hbm, dst_hbm, x_hbm, out_hbm, aggr_sh, db0, db1, ib0, ib1,
                  rows0, rows1, zbuf, abuf, xbuf, gs0, gs1, ss0, ss1, ds0, ds1):
    c = lax.axis_index("c")
    t = lax.axis_index("s")
    lo = c * HALF
    ebase = t * E_PER_T
    lanes = lax.iota(jnp.int32, 16)
    zero = jnp.zeros((16,), jnp.float32)

    for r in range(R_CH):
        for q in range(D // 16):
            zbuf[r, pl.ds(q * 16, 16)] = zero

    def init_chunk(i, carry):
        cid = t + i * NS

        @pl.when(cid < N_RCH)
        def _():
            off = pl.multiple_of(cid * R_CH, 8)
            pltpu.sync_copy(zbuf, aggr_sh.at[pl.ds(off, R_CH)])

        return carry

    lax.fori_loop(0, RCH_PER_T, init_chunk, 0)
    plsc.subcore_barrier()

    dbuf = (db0, db1)
    ibuf = (ib0, ib1)
    rows = (rows0, rows1)
    gsem = (gs0, gs1)
    ssem = (ss0, ss1)
    dsem = (ds0, ds1)

    def compute_idx(db, ib, jj):
        for q in range(S_CH // 16):
            d = db[pl.ds(q * 16, 16)] - lo
            m = (d >= 0) & (d < HALF)
            garb = HALF + ((lanes + jj + q) & (N_GARB - 1))
            ib[pl.ds(q * 16, 16)] = jnp.where(m, d, garb)

    def dst_slice(jj):
        return dst_hbm.at[pl.ds(pl.multiple_of(ebase + jj * S_CH, 8), S_CH)]

    pltpu.async_copy(dst_slice(0), dbuf[0], dsem[0])
    pltpu.async_copy(dst_slice(1), dbuf[1], dsem[1])

    def pair(j2, carry):
        for b in (0, 1):
            jj = j2 * 2 + b
            nb = 1 - b

            @pl.when(jj < S_NCH)
            def _():
                @pl.when(jj >= 2)
                def _():
                    pltpu.make_async_copy(
                        rows[b], aggr_sh.at[pl.ds(0, S_CH)], ssem[b]).wait()

                eoff = pl.multiple_of(ebase + jj * S_CH, 8)
                pltpu.async_copy(
                    h_hbm.at[pl.ds(eoff, S_CH)], rows[b], gsem[b])
                pltpu.make_async_copy(
                    dst_slice(0), dbuf[b], dsem[b]).wait()
                compute_idx(dbuf[b], ibuf[b], jj)

                @pl.when(jj + 2 < S_NCH)
                def _():
                    pltpu.async_copy(dst_slice(jj + 2), dbuf[b], dsem[b])

            @pl.when((jj >= 1) & (jj <= S_NCH))
            def _():
                pltpu.make_async_copy(
                    h_hbm.at[pl.ds(0, S_CH)], rows[nb], gsem[nb]).wait()
                pltpu.make_async_copy(
                    rows[nb], aggr_sh.at[ibuf[nb]], ssem[nb]).start(add=True)

        return carry

    lax.fori_loop(0, (S_NCH + 2) // 2, pair, 0)
    pltpu.make_async_copy(rows[0], aggr_sh.at[pl.ds(0, S_CH)], ssem[0]).wait()
    pltpu.make_async_copy(rows[1], aggr_sh.at[pl.ds(0, S_CH)], ssem[1]).wait()

    plsc.subcore_barrier()

    def out_chunk(i, carry):
        cid = t + i * NS

        @pl.when(cid < N_RCH)
        def _():
            off = pl.multiple_of(cid * R_CH, 8)
            goff = pl.multiple_of(lo + cid * R_CH, 8)
            pltpu.sync_copy(aggr_sh.at[pl.ds(off, R_CH)], abuf)
            pltpu.sync_copy(x_hbm.at[pl.ds(goff, R_CH)], xbuf)
            for r in range(R_CH):
                for q in range(D // 16):
                    sl = pl.ds(q * 16, 16)
                    abuf[r, sl] = abuf[r, sl] + xbuf[r, sl]
            pltpu.sync_copy(abuf, out_hbm.at[pl.ds(goff, R_CH)])

        return carry

    lax.fori_loop(0, RCH_PER_T, out_chunk, 0)


def _sc_scatter(h, dst, x_feat):
    k = pl.kernel(
        _scatter_body,
        out_type=jax.ShapeDtypeStruct((N_NODES, D), jnp.float32),
        mesh=_sc_mesh(),
        compiler_params=pltpu.CompilerParams(use_tc_tiling_on_sc=False),
        scratch_types=[
            pltpu.VMEM_SHARED((HALF + N_GARB, D), jnp.float32),
            pltpu.VMEM((S_CH,), jnp.int32),
            pltpu.VMEM((S_CH,), jnp.int32),
            pltpu.VMEM((S_CH,), jnp.int32),
            pltpu.VMEM((S_CH,), jnp.int32),
            pltpu.VMEM((S_CH, D), jnp.float32),
            pltpu.VMEM((S_CH, D), jnp.float32),
            pltpu.VMEM((R_CH, D), jnp.float32),
            pltpu.VMEM((R_CH, D), jnp.float32),
            pltpu.VMEM((R_CH, D), jnp.float32),
            pltpu.SemaphoreType.DMA,
            pltpu.SemaphoreType.DMA,
            pltpu.SemaphoreType.DMA,
            pltpu.SemaphoreType.DMA,
            pltpu.SemaphoreType.DMA,
            pltpu.SemaphoreType.DMA,
        ],
    )
    return k(h, dst, x_feat)


BE = 4000


def _edge_ffn_body(xg_ref, ea_ref, bs_ref, w_ref, b_ref, o_ref):
    xe = xg_ref[...] + ea_ref[...]
    z = jnp.dot(xe, w_ref[...], preferred_element_type=jnp.float32) + b_ref[...]
    g = 0.5 * z * (1.0 + lax.erf(z * np.float32(1.0 / np.sqrt(2.0))))
    o_ref[...] = g * bs_ref[...]


def _tc_edge_ffn(xg, edge_attr, bases, W_pre, b_pre):
    return pl.pallas_call(
        _edge_ffn_body,
        grid=(N_EDGES // BE,),
        in_specs=[
            pl.BlockSpec((BE, D), lambda i: (i, 0)),
            pl.BlockSpec((BE, D), lambda i: (i, 0)),
            pl.BlockSpec((BE, D), lambda i: (i, 0)),
            pl.BlockSpec((D, D), lambda i: (0, 0)),
            pl.BlockSpec((1, D), lambda i: (0, 0)),
        ],
        out_specs=pl.BlockSpec((BE, D), lambda i: (i, 0)),
        out_shape=jax.ShapeDtypeStruct((N_EDGES, D), jnp.float32),
    )(xg, edge_attr, bases, W_pre, b_pre.reshape(1, D))


def _node_ffn_body(x_ref, w1_ref, b1_ref, g1_ref, be1_ref, w2_ref, b2_ref,
                   g2_ref, be2_ref, o_ref):
    x = x_ref[...]
    y = jnp.dot(x, w1_ref[...], preferred_element_type=jnp.float32) + b1_ref[...]
    m = jnp.mean(y, axis=0, keepdims=True)
    v = jnp.mean((y - m) * (y - m), axis=0, keepdims=True)
    y = (y - m) * lax.rsqrt(v + 1e-5) * g1_ref[...] + be1_ref[...]
    y = jnp.maximum(y, 0.0)
    y = jnp.dot(y, w2_ref[...], preferred_element_type=jnp.float32) + b2_ref[...]
    m = jnp.mean(y, axis=0, keepdims=True)
    v = jnp.mean((y - m) * (y - m), axis=0, keepdims=True)
    y = (y - m) * lax.rsqrt(v + 1e-5) * g2_ref[...] + be2_ref[...]
    y = jnp.maximum(y, 0.0)
    o_ref[...] = x + y


def _tc_node_ffn(x, W1, b1, g1, be1, W2, b2, g2, be2):
    row = lambda a: a.reshape(1, D)
    return pl.pallas_call(
        _node_ffn_body,
        out_shape=jax.ShapeDtypeStruct((N_NODES, D), jnp.float32),
    )(x, W1, row(b1), row(g1), row(be1), W2, row(b2), row(g2), row(be2))


def kernel(x_feat, edge_index, edge_attr, bases, W_pre, b_pre, W1, b1, g1, be1,
           W2, b2, g2, be2):
    src = edge_index[0]
    dst = edge_index[1]
    xg = _sc_gather(x_feat, src)
    h = _tc_edge_ffn(xg, edge_attr, bases, W_pre, b_pre)
    x = _sc_scatter(h, dst, x_feat)
    return _tc_node_ffn(x, W1, b1, g1, be1, W2, b2, g2, be2)
